# baseline (speedup 1.0000x reference)
import functools

import jax
import jax.numpy as jnp
from jax import lax
from jax.experimental import pallas as pl
from jax.experimental.pallas import tpu as pltpu
from jax.experimental.pallas import tpu_sc as plsc


@functools.lru_cache(maxsize=None)
def _make_gather(n_total: int, vocab: int, dmodel: int):
  info = plsc.get_sparse_core_info()
  nw = info.num_cores * info.num_subcores
  assert n_total % nw == 0
  per_w = n_total // nw
  chunk = 800
  assert per_w % chunk == 0 and chunk % 8 == 0
  n_chunks = per_w // chunk

  mesh = plsc.VectorSubcoreMesh(core_axis_name="c", subcore_axis_name="s")

  @functools.partial(
      pl.kernel,
      mesh=mesh,
      compiler_params=pltpu.CompilerParams(use_tc_tiling_on_sc=False),
      out_type=jax.ShapeDtypeStruct((n_total, dmodel), jnp.float32),
      scratch_types=[
          pltpu.VMEM((chunk,), jnp.int32),
          pltpu.VMEM((chunk, dmodel), jnp.float32),
          pltpu.SemaphoreType.DMA,
      ],
  )
  def k(idx_hbm, table_hbm, out_hbm, idx_v, rows_v, sem):
    wid = lax.axis_index("s") * info.num_cores + lax.axis_index("c")
    base = wid * per_w

    def body(g, carry):
      off = base + g * chunk
      pltpu.sync_copy(idx_hbm.at[pl.ds(off, chunk)], idx_v)
      pltpu.async_copy(table_hbm.at[idx_v], rows_v, sem).wait()
      pltpu.sync_copy(rows_v, out_hbm.at[pl.ds(off, chunk)])
      return carry

    lax.fori_loop(0, n_chunks, body, 0)

  return k


def kernel(x, weight):
  b, s = x.shape
  vocab, dmodel = weight.shape
  flat_idx = x.reshape(-1).astype(jnp.int32)
  out = _make_gather(b * s, vocab, dmodel)(flat_idx, weight)
  return out.reshape(b, s, dmodel)


# speedup vs baseline: 1.0238x; 1.0238x over previous
import functools

import jax
import jax.numpy as jnp
from jax import lax
from jax.experimental import pallas as pl
from jax.experimental.pallas import tpu as pltpu
from jax.experimental.pallas import tpu_sc as plsc


@functools.lru_cache(maxsize=None)
def _make_gather(n_total: int, vocab: int, dmodel: int):
  info = plsc.get_sparse_core_info()
  nw = info.num_cores * info.num_subcores
  assert n_total % nw == 0
  per_w = n_total // nw
  chunk = 800
  assert per_w % chunk == 0 and chunk % 8 == 0
  n_chunks = per_w // chunk

  mesh = plsc.VectorSubcoreMesh(core_axis_name="c", subcore_axis_name="s")

  @functools.partial(
      pl.kernel,
      mesh=mesh,
      compiler_params=pltpu.CompilerParams(use_tc_tiling_on_sc=False),
      out_type=jax.ShapeDtypeStruct((n_total, dmodel), jnp.float32),
      scratch_types=[
          pltpu.VMEM((per_w,), jnp.int32),
          pltpu.VMEM((2, chunk, dmodel), jnp.float32),
          pltpu.SemaphoreType.DMA,
          pltpu.SemaphoreType.DMA,
          pltpu.SemaphoreType.DMA,
          pltpu.SemaphoreType.DMA,
      ],
  )
  def k(idx_hbm, table_hbm, out_hbm, idx_v, rows_v, sg0, sg1, sw0, sw1):
    wid = lax.axis_index("s") * info.num_cores + lax.axis_index("c")
    base = wid * per_w
    pltpu.sync_copy(idx_hbm.at[pl.ds(base, per_w)], idx_v)

    sg = (sg0, sg1)
    sw = (sw0, sw1)

    def start_gather(g):
      return pltpu.async_copy(
          table_hbm.at[idx_v.at[pl.ds(g * chunk, chunk)]],
          rows_v.at[g % 2],
          sg[g % 2],
      )

    def start_wb(g):
      return pltpu.async_copy(
          rows_v.at[g % 2],
          out_hbm.at[pl.ds(base + g * chunk, chunk)],
          sw[g % 2],
      )

    gh = [None] * n_chunks
    wh = [None] * n_chunks
    gh[0] = start_gather(0)
    for g in range(n_chunks):
      if g >= 2:
        wh[g - 2].wait()
      if g >= 1:
        gh[g] = start_gather(g)
        gh[g - 1].wait()
        wh[g - 1] = start_wb(g - 1)
    gh[n_chunks - 1].wait()
    wh[n_chunks - 1] = start_wb(n_chunks - 1)
    wh[n_chunks - 2].wait()
    wh[n_chunks - 1].wait()

  return k


def kernel(x, weight):
  b, s = x.shape
  vocab, dmodel = weight.shape
  flat_idx = x.reshape(-1).astype(jnp.int32)
  out = _make_gather(b * s, vocab, dmodel)(flat_idx, weight)
  return out.reshape(b, s, dmodel)
